# initial kernel scaffold (unmeasured)
import jax
import jax.numpy as jnp
from jax import lax
from jax.experimental import pallas as pl
from jax.experimental.pallas import tpu as pltpu


def kernel(O, Wo):
    B, S, H, D = O.shape
    K = H * D
    N = Wo.shape[1]
    S_half = S // 2
    M = B * S_half

    O = O.reshape(B, S, K).astype(jnp.bfloat16)
    Wo = Wo.astype(jnp.bfloat16)

    CH = 512

    def body(o_ref, w_ref, out_ref, send_buf, recv_buf, send_sem, recv_sem):
        my_x = lax.axis_index("x")
        my_y = lax.axis_index("y")
        peer = (my_x, 1 - my_y)

        barrier_sem = pltpu.get_barrier_semaphore()
        pl.semaphore_signal(
            barrier_sem, inc=1, device_id=peer,
            device_id_type=pltpu.DeviceIdType.MESH,
        )
        pl.semaphore_wait(barrier_sem, 1)

        theirs = o_ref[:, pl.ds((1 - my_y) * S_half, S_half), :].reshape(M, K)
        for nc in range(0, N, CH):
            send_buf[:, nc:nc + CH] = jnp.dot(
                theirs, w_ref[:, nc:nc + CH],
                preferred_element_type=jnp.float32,
            ).astype(jnp.bfloat16)

        rdma = pltpu.make_async_remote_copy(
            src_ref=send_buf,
            dst_ref=recv_buf,
            send_sem=send_sem,
            recv_sem=recv_sem,
            device_id=peer,
            device_id_type=pltpu.DeviceIdType.MESH,
        )
        rdma.start()

        mine = o_ref[:, pl.ds(my_y * S_half, S_half), :].reshape(M, K)
        for nc in range(0, N, CH):
            out_ref[:, nc:nc + CH] = jnp.dot(
                mine, w_ref[:, nc:nc + CH],
                preferred_element_type=jnp.float32,
            )

        rdma.wait()
        out_ref[...] += recv_buf[...].astype(jnp.float32)

    out = pl.pallas_call(
        body,
        out_shape=jax.ShapeDtypeStruct((M, N), jnp.float32),
        in_specs=[
            pl.BlockSpec(memory_space=pltpu.VMEM),
            pl.BlockSpec(memory_space=pltpu.VMEM),
        ],
        out_specs=pl.BlockSpec(memory_space=pltpu.VMEM),
        scratch_shapes=[
            pltpu.VMEM((M, N), jnp.bfloat16),
            pltpu.VMEM((M, N), jnp.bfloat16),
            pltpu.SemaphoreType.DMA,
            pltpu.SemaphoreType.DMA,
        ],
        compiler_params=pltpu.CompilerParams(collective_id=0),
    )(O, Wo)
    return out.reshape(B, S_half, N)


# baseline (device time: 277382 ns/iter reference)
import jax
import jax.numpy as jnp
from jax import lax
from jax.experimental import pallas as pl
from jax.experimental.pallas import tpu as pltpu

NCH = 8


def kernel(O, Wo):
    B, S, H, D = O.shape
    K = H * D
    N = Wo.shape[1]
    S_half = S // 2
    M = B * S_half
    CH = N // NCH

    my_y = lax.axis_index("y")
    Or = O.reshape(B, S, K).astype(jnp.bfloat16)
    o_mine = lax.dynamic_slice_in_dim(Or, my_y * S_half, S_half, 1).reshape(M, K)
    o_theirs = lax.dynamic_slice_in_dim(
        Or, (1 - my_y) * S_half, S_half, 1
    ).reshape(M, K)
    Wo = Wo.astype(jnp.bfloat16)

    def body(
        mine_ref, theirs_ref, w_hbm,
        out_hbm,
        w_buf, send_buf, recv_buf, stage_buf,
        w_sems, send_sems, recv_sems, store_sems,
    ):
        my_x = lax.axis_index("x")
        peer = (my_x, 1 - lax.axis_index("y"))

        barrier_sem = pltpu.get_barrier_semaphore()
        pl.semaphore_signal(
            barrier_sem, inc=1, device_id=peer,
            device_id_type=pltpu.DeviceIdType.MESH,
        )
        pl.semaphore_wait(barrier_sem, 1)

        def w_copy(c, slot):
            return pltpu.make_async_copy(
                w_hbm.at[:, c * CH:(c + 1) * CH], w_buf.at[slot], w_sems.at[slot]
            )

        w_copy(0, 0).start()
        rdmas = []
        for c in range(NCH):
            slot = c % 2
            if c + 1 < NCH:
                w_copy(c + 1, (c + 1) % 2).start()
            w_copy(c, slot).wait()
            if c >= 2:
                rdmas[c - 2].wait_send()
            send_buf[slot] = jnp.dot(
                theirs_ref[...], w_buf[slot],
                preferred_element_type=jnp.float32,
            ).astype(jnp.bfloat16)
            rdma = pltpu.make_async_remote_copy(
                src_ref=send_buf.at[slot],
                dst_ref=recv_buf.at[c],
                send_sem=send_sems.at[slot],
                recv_sem=recv_sems.at[c],
                device_id=peer,
                device_id_type=pltpu.DeviceIdType.MESH,
            )
            rdma.start()
            rdmas.append(rdma)

        w_copy(0, 0).start()
        stores = []
        for c in range(NCH):
            slot = c % 2
            if c + 1 < NCH:
                w_copy(c + 1, (c + 1) % 2).start()
            w_copy(c, slot).wait()
            mine_c = jnp.dot(
                mine_ref[...], w_buf[slot], preferred_element_type=jnp.float32
            )
            rdmas[c].wait_recv()
            total = mine_c + recv_buf[c].astype(jnp.float32)
            if c >= 2:
                stores[c - 2].wait()
            stage_buf[slot] = total.astype(jnp.bfloat16)
            store = pltpu.make_async_copy(
                stage_buf.at[slot],
                out_hbm.at[:, c * CH:(c + 1) * CH],
                store_sems.at[slot],
            )
            store.start()
            stores.append(store)

        rdmas[NCH - 2].wait_send()
        rdmas[NCH - 1].wait_send()
        stores[NCH - 2].wait()
        stores[NCH - 1].wait()

    out = pl.pallas_call(
        body,
        out_shape=jax.ShapeDtypeStruct((M, N), jnp.bfloat16),
        in_specs=[
            pl.BlockSpec(memory_space=pltpu.VMEM),
            pl.BlockSpec(memory_space=pltpu.VMEM),
            pl.BlockSpec(memory_space=pl.ANY),
        ],
        out_specs=pl.BlockSpec(memory_space=pl.ANY),
        scratch_shapes=[
            pltpu.VMEM((2, K, CH), jnp.bfloat16),
            pltpu.VMEM((2, M, CH), jnp.bfloat16),
            pltpu.VMEM((NCH, M, CH), jnp.bfloat16),
            pltpu.VMEM((2, M, CH), jnp.bfloat16),
            pltpu.SemaphoreType.DMA((2,)),
            pltpu.SemaphoreType.DMA((2,)),
            pltpu.SemaphoreType.DMA((NCH,)),
            pltpu.SemaphoreType.DMA((2,)),
        ],
        compiler_params=pltpu.CompilerParams(
            collective_id=0,
            vmem_limit_bytes=60 * 1024 * 1024,
        ),
    )(o_mine, o_theirs, Wo)
    return out.reshape(B, S_half, N)


# device time: 253609 ns/iter; 1.0937x vs baseline; 1.0937x over previous
import jax
import jax.numpy as jnp
from jax import lax
from jax.experimental import pallas as pl
from jax.experimental.pallas import tpu as pltpu

CHUNKS = [512] * 8
NCH = len(CHUNKS)
CMAX = max(CHUNKS)


def kernel(O, Wo):
    B, S, H, D = O.shape
    K = H * D
    N = Wo.shape[1]
    S_half = S // 2
    M = B * S_half
    assert sum(CHUNKS) == N
    OFFS = [sum(CHUNKS[:i]) for i in range(NCH)]

    my_y = lax.axis_index("y")

    def half(start):
        return lax.dynamic_slice_in_dim(O, start, S_half, 1).astype(jnp.bfloat16)

    o_mine = half(my_y * S_half)
    o_theirs = half((1 - my_y) * S_half)
    Wo = Wo.astype(jnp.bfloat16)

    def body(
        mine_ref, theirs_ref, w_hbm,
        out_ref,
        w_buf, send_buf,
        w_sems, send_sems, recv_sems,
    ):
        my_x = lax.axis_index("x")
        peer = (my_x, 1 - lax.axis_index("y"))

        barrier_sem = pltpu.get_barrier_semaphore()
        pl.semaphore_signal(
            barrier_sem, inc=1, device_id=peer,
            device_id_type=pltpu.DeviceIdType.MESH,
        )
        pl.semaphore_wait(barrier_sem, 1)

        def w_copy(c, slot):
            sz = CHUNKS[c]
            return pltpu.make_async_copy(
                w_hbm.at[:, OFFS[c]:OFFS[c] + sz],
                w_buf.at[slot, :, 0:sz],
                w_sems.at[slot],
            )

        theirs = theirs_ref[...].reshape(M, K)
        w_copy(0, 0).start()
        rdmas = []
        for c in range(NCH):
            slot = c % 2
            sz = CHUNKS[c]
            if c + 1 < NCH:
                w_copy(c + 1, (c + 1) % 2).start()
            w_copy(c, slot).wait()
            if c >= 2:
                rdmas[c - 2].wait_send()
            send_buf[slot, :, :, 0:sz] = jnp.dot(
                theirs, w_buf[slot, :, 0:sz],
                preferred_element_type=jnp.float32,
            ).astype(jnp.bfloat16).reshape(B, S_half, sz)
            rdma = pltpu.make_async_remote_copy(
                src_ref=send_buf.at[slot, :, :, 0:sz],
                dst_ref=out_ref.at[:, :, OFFS[c]:OFFS[c] + sz],
                send_sem=send_sems.at[slot],
                recv_sem=recv_sems.at[c],
                device_id=peer,
                device_id_type=pltpu.DeviceIdType.MESH,
            )
            rdma.start()
            rdmas.append(rdma)

        mine = mine_ref[...].reshape(M, K)
        w_copy(0, 0).start()
        for c in range(NCH):
            slot = c % 2
            sz = CHUNKS[c]
            if c + 1 < NCH:
                w_copy(c + 1, (c + 1) % 2).start()
            w_copy(c, slot).wait()
            mine_c = jnp.dot(
                mine, w_buf[slot, :, 0:sz], preferred_element_type=jnp.float32
            ).reshape(B, S_half, sz)
            rdmas[c].wait_recv()
            cols = slice(OFFS[c], OFFS[c] + sz)
            out_ref[:, :, cols] = (
                mine_c + out_ref[:, :, cols].astype(jnp.float32)
            ).astype(jnp.bfloat16)

        rdmas[NCH - 2].wait_send()
        rdmas[NCH - 1].wait_send()

    out = pl.pallas_call(
        body,
        out_shape=jax.ShapeDtypeStruct((B, S_half, N), jnp.bfloat16),
        in_specs=[
            pl.BlockSpec(memory_space=pltpu.VMEM),
            pl.BlockSpec(memory_space=pltpu.VMEM),
            pl.BlockSpec(memory_space=pl.ANY),
        ],
        out_specs=pl.BlockSpec(memory_space=pltpu.VMEM),
        scratch_shapes=[
            pltpu.VMEM((2, K, CMAX), jnp.bfloat16),
            pltpu.VMEM((2, B, S_half, CMAX), jnp.bfloat16),
            pltpu.SemaphoreType.DMA((2,)),
            pltpu.SemaphoreType.DMA((2,)),
            pltpu.SemaphoreType.DMA((NCH,)),
        ],
        compiler_params=pltpu.CompilerParams(
            collective_id=0,
            vmem_limit_bytes=62 * 1024 * 1024,
        ),
    )(o_mine, o_theirs, Wo)
    return out
